# baseline (speedup 1.0000x reference)
import functools

import jax
import jax.numpy as jnp
from jax import lax
from jax.experimental import pallas as pl
from jax.experimental.pallas import tpu as pltpu
from jax.experimental.pallas import tpu_sc as plsc

N = 10000
E = 320000
D = 128

NC = 2
NS = 16
NW = NC * NS
NPAD = 10240
SPT = NPAD // NS
EPW = E // NW
SUB = 80
RPW = EPW // SUB


def _mesh():
    return plsc.VectorSubcoreMesh(
        core_axis_name="c", subcore_axis_name="s", num_cores=NC, num_subcores=NS
    )


def _sc_degrees(src, dst):
    @functools.partial(
        pl.kernel,
        mesh=_mesh(),
        out_type=(
            jax.ShapeDtypeStruct((NC * NPAD,), jnp.float32),
            jax.ShapeDtypeStruct((NC * NPAD,), jnp.float32),
        ),
        scratch_types=[
            pltpu.VMEM_SHARED((NPAD,), jnp.float32),
            pltpu.VMEM_SHARED((NPAD,), jnp.float32),
            pltpu.VMEM((SUB,), jnp.float32),
            pltpu.VMEM((SUB,), jnp.float32),
            pltpu.VMEM((4 * SUB,), jnp.int32),
            pltpu.VMEM((4 * SUB,), jnp.int32),
            pltpu.VMEM((SPT,), jnp.float32),
            pltpu.SemaphoreType.DMA,
            pltpu.SemaphoreType.DMA,
        ],
    )
    def k(src_h, dst_h, dego_h, degi_h, ho_sh, hi_sh, ones_v, zo_v, sq, dq,
          zv, ld_sem, sc_sem):
        c = lax.axis_index("c")
        s = lax.axis_index("s")
        wid = s * NC + c
        ebase = wid * EPW

        def fill1(i, carry):
            ones_v[pl.ds(i * 16, 16)] = jnp.ones((16,), jnp.float32)
            zo_v[pl.ds(i * 16, 16)] = jnp.zeros((16,), jnp.float32)
            return carry

        lax.fori_loop(0, SUB // 16, fill1, 0)

        def fillq(i, carry):
            sq[pl.ds(i * 16, 16)] = jnp.zeros((16,), jnp.int32)
            dq[pl.ds(i * 16, 16)] = jnp.zeros((16,), jnp.int32)
            return carry

        lax.fori_loop(0, (4 * SUB) // 16, fillq, 0)

        def fill0(i, carry):
            zv[pl.ds(i * 16, 16)] = jnp.zeros((16,), jnp.float32)
            return carry

        lax.fori_loop(0, SPT // 16, fill0, 0)

        pltpu.sync_copy(zv, ho_sh.at[pl.ds(s * SPT, SPT)])
        pltpu.sync_copy(zv, hi_sh.at[pl.ds(s * SPT, SPT)])
        plsc.subcore_barrier()

        def q4(t):
            return lax.rem(t, 4) * SUB

        def lbase(t):
            return ebase + jnp.minimum(t, RPW - 1) * SUB

        def issue_loads(t):
            pltpu.async_copy(src_h.at[pl.ds(lbase(t), SUB)],
                             sq.at[pl.ds(q4(t), SUB)], ld_sem)
            pltpu.async_copy(dst_h.at[pl.ds(lbase(t), SUB)],
                             dq.at[pl.ds(q4(t), SUB)], ld_sem)

        def wait_loads(t):
            pltpu.make_async_copy(src_h.at[pl.ds(lbase(t), SUB)],
                                  sq.at[pl.ds(q4(t), SUB)], ld_sem).wait()
            pltpu.make_async_copy(dst_h.at[pl.ds(lbase(t), SUB)],
                                  dq.at[pl.ds(q4(t), SUB)], ld_sem).wait()

        def drain_scatter(t):
            pltpu.make_async_copy(
                ones_v, ho_sh.at[sq.at[pl.ds(q4(t), SUB)]], sc_sem).wait()
            pltpu.make_async_copy(
                ones_v, hi_sh.at[dq.at[pl.ds(q4(t), SUB)]], sc_sem).wait()

        for slot in (2, 3):
            pltpu.async_copy(zo_v, ho_sh.at[sq.at[pl.ds(slot * SUB, SUB)]],
                             sc_sem, add=True)
            pltpu.async_copy(zo_v, hi_sh.at[dq.at[pl.ds(slot * SUB, SUB)]],
                             sc_sem, add=True)
        issue_loads(0)
        issue_loads(1)

        def hbody(bi, carry):
            drain_scatter(bi + 2)
            issue_loads(bi + 2)
            wait_loads(bi)
            qb = q4(bi)
            pltpu.async_copy(ones_v, ho_sh.at[sq.at[pl.ds(qb, SUB)]],
                             sc_sem, add=True)
            pltpu.async_copy(ones_v, hi_sh.at[dq.at[pl.ds(qb, SUB)]],
                             sc_sem, add=True)
            return carry

        lax.fori_loop(0, RPW, hbody, 0)

        for t in (RPW - 2, RPW - 1):
            drain_scatter(t)
        for t in (RPW, RPW + 1):
            wait_loads(t)
        plsc.subcore_barrier()

        for sh, outh in ((ho_sh, dego_h), (hi_sh, degi_h)):
            pltpu.sync_copy(sh.at[pl.ds(s * SPT, SPT)], zv)
            pltpu.sync_copy(zv, outh.at[pl.ds(c * NPAD + s * SPT, SPT)])

    return k(src, dst)


def _tc_scales(po0, po1, pi0, pi1):
    def body(a_ref, b_ref, c_ref, d_ref, so_ref, si_ref):
        so_ref[...] = lax.rsqrt(jnp.maximum(a_ref[...] + b_ref[...], 1.0))
        si_ref[...] = lax.rsqrt(jnp.maximum(c_ref[...] + d_ref[...], 1.0))

    return pl.pallas_call(
        body,
        out_shape=(
            jax.ShapeDtypeStruct((NPAD,), jnp.float32),
            jax.ShapeDtypeStruct((NPAD,), jnp.float32),
        ),
    )(po0, po1, pi0, pi1)


def _tc_matmul(x, W):
    BR = 1000

    def body(x_ref, w_ref, h_ref):
        h_ref[...] = jnp.dot(
            x_ref[...], w_ref[...], preferred_element_type=jnp.float32
        )

    return pl.pallas_call(
        body,
        grid=(N // BR,),
        in_specs=[
            pl.BlockSpec((BR, D), lambda i: (i, 0)),
            pl.BlockSpec((D, D), lambda i: (0, 0)),
        ],
        out_specs=pl.BlockSpec((BR, D), lambda i: (i, 0)),
        out_shape=jax.ShapeDtypeStruct((N, D), jnp.float32),
    )(x, W)


def _sc_gather_scatter(h, src, dst, w, s_out, s_in):
    @functools.partial(
        pl.kernel,
        mesh=_mesh(),
        out_type=jax.ShapeDtypeStruct((NC, NPAD, D), jnp.float32),
        scratch_types=[
            pltpu.VMEM_SHARED((NPAD, D), jnp.float32),
            pltpu.VMEM((3 * SUB, D), jnp.float32),
            pltpu.VMEM((4 * SUB,), jnp.int32),
            pltpu.VMEM((4 * SUB,), jnp.int32),
            pltpu.VMEM((4 * SUB,), jnp.float32),
            pltpu.VMEM((4 * SUB,), jnp.float32),
            pltpu.VMEM((4 * SUB,), jnp.float32),
            pltpu.VMEM((SUB, D), jnp.float32),
            pltpu.SemaphoreType.DMA,
            pltpu.SemaphoreType.DMA,
            pltpu.SemaphoreType.DMA,
        ],
    )
    def k(h_h, src_h, dst_h, w_h, so_h, si_h, out_h,
          acc_sh, rows_v, sq, dq, wq, g1q, g2q, zv, ld_sem, g_sem, sc_sem):
        c = lax.axis_index("c")
        s = lax.axis_index("s")
        wid = s * NC + c
        ebase = wid * EPW

        def fillz(i, carry):
            for cc in range(8):
                zv[i, pl.ds(cc * 16, 16)] = jnp.zeros((16,), jnp.float32)
            return carry

        lax.fori_loop(0, SUB, fillz, 0)

        def fillq(i, carry):
            sq[pl.ds(i * 16, 16)] = jnp.zeros((16,), jnp.int32)
            dq[pl.ds(i * 16, 16)] = jnp.zeros((16,), jnp.int32)
            return carry

        lax.fori_loop(0, (4 * SUB) // 16, fillq, 0)

        zs = [
            pltpu.async_copy(zv, acc_sh.at[pl.ds(s * SPT + j * SUB, SUB)],
                             ld_sem)
            for j in range(SPT // SUB)
        ]
        for z in zs:
            z.wait()
        plsc.subcore_barrier()

        def q4(t):
            return lax.rem(t, 4) * SUB

        def r3(t):
            return lax.rem(t, 3) * SUB

        def lbase(t):
            return ebase + jnp.minimum(t, RPW - 1) * SUB

        def issue_loads(t):
            pltpu.async_copy(src_h.at[pl.ds(lbase(t), SUB)],
                             sq.at[pl.ds(q4(t), SUB)], ld_sem)
            pltpu.async_copy(dst_h.at[pl.ds(lbase(t), SUB)],
                             dq.at[pl.ds(q4(t), SUB)], ld_sem)
            pltpu.async_copy(w_h.at[pl.ds(lbase(t), SUB)],
                             wq.at[pl.ds(q4(t), SUB)], ld_sem)

        def wait_loads(t):
            pltpu.make_async_copy(src_h.at[pl.ds(lbase(t), SUB)],
                                  sq.at[pl.ds(q4(t), SUB)], ld_sem).wait()
            pltpu.make_async_copy(dst_h.at[pl.ds(lbase(t), SUB)],
                                  dq.at[pl.ds(q4(t), SUB)], ld_sem).wait()
            pltpu.make_async_copy(w_h.at[pl.ds(lbase(t), SUB)],
                                  wq.at[pl.ds(q4(t), SUB)], ld_sem).wait()

        def issue_gather(t):
            pltpu.async_copy(h_h.at[sq.at[pl.ds(q4(t), SUB)]],
                             rows_v.at[pl.ds(r3(t), SUB)], g_sem)
            pltpu.async_copy(so_h.at[sq.at[pl.ds(q4(t), SUB)]],
                             g1q.at[pl.ds(q4(t), SUB)], g_sem)
            pltpu.async_copy(si_h.at[dq.at[pl.ds(q4(t), SUB)]],
                             g2q.at[pl.ds(q4(t), SUB)], g_sem)

        def wait_gather(t):
            pltpu.make_async_copy(h_h.at[sq.at[pl.ds(q4(t), SUB)]],
                                  rows_v.at[pl.ds(r3(t), SUB)], g_sem).wait()
            pltpu.make_async_copy(so_h.at[sq.at[pl.ds(q4(t), SUB)]],
                                  g1q.at[pl.ds(q4(t), SUB)], g_sem).wait()
            pltpu.make_async_copy(si_h.at[dq.at[pl.ds(q4(t), SUB)]],
                                  g2q.at[pl.ds(q4(t), SUB)], g_sem).wait()

        def wait_scatter(t):
            pltpu.make_async_copy(
                rows_v.at[pl.ds(r3(t), SUB)],
                acc_sh.at[dq.at[pl.ds(q4(t), SUB)]], sc_sem).wait()

        for slot in (2, 3):
            pltpu.async_copy(zv, acc_sh.at[dq.at[pl.ds(slot * SUB, SUB)]],
                             sc_sem, add=True)
        issue_loads(0)
        issue_loads(1)
        wait_loads(0)
        issue_gather(0)

        dnums = lax.GatherDimensionNumbers(
            offset_dims=(), collapsed_slice_dims=(0,), start_index_map=(0,)
        )

        def body(bi, carry):
            pltpu.make_async_copy(
                rows_v.at[pl.ds(r3(bi + 1), SUB)],
                acc_sh.at[dq.at[pl.ds(q4(bi + 2), SUB)]],
                sc_sem).wait()
            wait_loads(bi + 1)
            issue_gather(bi + 1)
            issue_loads(bi + 2)
            wait_gather(bi)
            rb = r3(bi)
            qb = q4(bi)

            def mul(g, carry2):
                wsl = pl.ds(qb + g * 16, 16)
                wchunk = wq[wsl] * g1q[wsl] * g2q[wsl]
                for lane in range(16):
                    wv = lax.gather(
                        wchunk,
                        jnp.full((16, 1), lane, jnp.int32),
                        dnums,
                        (1,),
                        mode=lax.GatherScatterMode.PROMISE_IN_BOUNDS,
                    )
                    jj = rb + g * 16 + lane
                    for cc in range(8):
                        sl = pl.ds(cc * 16, 16)
                        rows_v[jj, sl] = rows_v[jj, sl] * wv
                return carry2

            lax.fori_loop(0, SUB // 16, mul, 0)

            pltpu.async_copy(rows_v.at[pl.ds(rb, SUB)],
                             acc_sh.at[dq.at[pl.ds(qb, SUB)]],
                             sc_sem, add=True)
            return carry

        lax.fori_loop(0, RPW, body, 0)

        for t in (RPW - 2, RPW - 1):
            wait_scatter(t)
        wait_gather(RPW)
        wait_loads(RPW + 1)
        plsc.subcore_barrier()
        for j in range(SPT // 160):
            pltpu.sync_copy(
                acc_sh.at[pl.ds(s * SPT + j * 160, 160)],
                rows_v.at[pl.ds(0, 160)],
            )
            pltpu.sync_copy(
                rows_v.at[pl.ds(0, 160)],
                out_h.at[c, pl.ds(s * SPT + j * 160, 160)],
            )

    return k(h, src, dst, w, s_out, s_in)


def _tc_combine(p, b2):
    BR = 1000

    def body(p_ref, b_ref, o_ref):
        o_ref[...] = p_ref[0] + p_ref[1] + b_ref[...]

    return pl.pallas_call(
        body,
        grid=(N // BR,),
        in_specs=[
            pl.BlockSpec((2, BR, D), lambda i: (0, i, 0)),
            pl.BlockSpec((1, D), lambda i: (0, 0)),
        ],
        out_specs=pl.BlockSpec((BR, D), lambda i: (i, 0)),
        out_shape=jax.ShapeDtypeStruct((N, D), jnp.float32),
    )(p, b2)


def kernel(x, edge_index, edge_weight, W, b):
    src = edge_index[0].astype(jnp.int32)
    dst = edge_index[1].astype(jnp.int32)
    w = edge_weight.astype(jnp.float32)

    h = _tc_matmul(x, W)
    dego_p, degi_p = _sc_degrees(src, dst)
    dego_p = dego_p.reshape(NC, NPAD)
    degi_p = degi_p.reshape(NC, NPAD)
    s_out, s_in = _tc_scales(dego_p[0], dego_p[1], degi_p[0], degi_p[1])
    partials = _sc_gather_scatter(h, src, dst, w, s_out, s_in)
    out = _tc_combine(partials, b.reshape(1, D))
    return out


# speedup vs baseline: 1.2493x; 1.2493x over previous
import functools

import jax
import jax.numpy as jnp
from jax import lax
from jax.experimental import pallas as pl
from jax.experimental.pallas import tpu as pltpu
from jax.experimental.pallas import tpu_sc as plsc

N = 10000
E = 320000
D = 128

NC = 2
NS = 16
NW = NC * NS
NPAD = 10240
SPT = NPAD // NS
EPW = E // NW
SUB = 80
RPW = EPW // SUB


def _mesh():
    return plsc.VectorSubcoreMesh(
        core_axis_name="c", subcore_axis_name="s", num_cores=NC, num_subcores=NS
    )


def _sc_degrees(ei):
    @functools.partial(
        pl.kernel,
        mesh=_mesh(),
        out_type=(
            jax.ShapeDtypeStruct((NC * NPAD,), jnp.float32),
            jax.ShapeDtypeStruct((NC * NPAD,), jnp.float32),
        ),
        scratch_types=[
            pltpu.VMEM_SHARED((NPAD,), jnp.float32),
            pltpu.VMEM_SHARED((NPAD,), jnp.float32),
            pltpu.VMEM((SUB,), jnp.float32),
            pltpu.VMEM((SUB,), jnp.float32),
            pltpu.VMEM((4 * SUB,), jnp.int32),
            pltpu.VMEM((4 * SUB,), jnp.int32),
            pltpu.VMEM((SPT,), jnp.float32),
            pltpu.SemaphoreType.DMA,
            pltpu.SemaphoreType.DMA,
        ],
    )
    def k(ei_h, dego_h, degi_h, ho_sh, hi_sh, ones_v, zo_v, sq, dq,
          zv, ld_sem, sc_sem):
        c = lax.axis_index("c")
        s = lax.axis_index("s")
        wid = s * NC + c
        ebase = wid * EPW

        def fill1(i, carry):
            ones_v[pl.ds(i * 16, 16)] = jnp.ones((16,), jnp.float32)
            zo_v[pl.ds(i * 16, 16)] = jnp.zeros((16,), jnp.float32)
            return carry

        lax.fori_loop(0, SUB // 16, fill1, 0)

        def fillq(i, carry):
            sq[pl.ds(i * 16, 16)] = jnp.zeros((16,), jnp.int32)
            dq[pl.ds(i * 16, 16)] = jnp.zeros((16,), jnp.int32)
            return carry

        lax.fori_loop(0, (4 * SUB) // 16, fillq, 0)

        def fill0(i, carry):
            zv[pl.ds(i * 16, 16)] = jnp.zeros((16,), jnp.float32)
            return carry

        lax.fori_loop(0, SPT // 16, fill0, 0)

        pltpu.sync_copy(zv, ho_sh.at[pl.ds(s * SPT, SPT)])
        pltpu.sync_copy(zv, hi_sh.at[pl.ds(s * SPT, SPT)])
        plsc.subcore_barrier()

        def q4(t):
            return lax.rem(t, 4) * SUB

        def lbase(t):
            return ebase + jnp.minimum(t, RPW - 1) * SUB

        def issue_loads(t):
            pltpu.async_copy(ei_h.at[pl.ds(lbase(t), SUB)],
                             sq.at[pl.ds(q4(t), SUB)], ld_sem)
            pltpu.async_copy(ei_h.at[pl.ds(E + lbase(t), SUB)],
                             dq.at[pl.ds(q4(t), SUB)], ld_sem)

        def wait_loads(t):
            pltpu.make_async_copy(ei_h.at[pl.ds(lbase(t), SUB)],
                                  sq.at[pl.ds(q4(t), SUB)], ld_sem).wait()
            pltpu.make_async_copy(ei_h.at[pl.ds(E + lbase(t), SUB)],
                                  dq.at[pl.ds(q4(t), SUB)], ld_sem).wait()

        def drain_scatter(t):
            pltpu.make_async_copy(
                ones_v, ho_sh.at[sq.at[pl.ds(q4(t), SUB)]], sc_sem).wait()
            pltpu.make_async_copy(
                ones_v, hi_sh.at[dq.at[pl.ds(q4(t), SUB)]], sc_sem).wait()

        for slot in (2, 3):
            pltpu.async_copy(zo_v, ho_sh.at[sq.at[pl.ds(slot * SUB, SUB)]],
                             sc_sem, add=True)
            pltpu.async_copy(zo_v, hi_sh.at[dq.at[pl.ds(slot * SUB, SUB)]],
                             sc_sem, add=True)
        issue_loads(0)
        issue_loads(1)

        def hbody(bi, carry):
            drain_scatter(bi + 2)
            issue_loads(bi + 2)
            wait_loads(bi)
            qb = q4(bi)
            pltpu.async_copy(ones_v, ho_sh.at[sq.at[pl.ds(qb, SUB)]],
                             sc_sem, add=True)
            pltpu.async_copy(ones_v, hi_sh.at[dq.at[pl.ds(qb, SUB)]],
                             sc_sem, add=True)
            return carry

        lax.fori_loop(0, RPW, hbody, 0)

        for t in (RPW - 2, RPW - 1):
            drain_scatter(t)
        for t in (RPW, RPW + 1):
            wait_loads(t)
        plsc.subcore_barrier()

        for sh, outh in ((ho_sh, dego_h), (hi_sh, degi_h)):
            pltpu.sync_copy(sh.at[pl.ds(s * SPT, SPT)], zv)
            pltpu.sync_copy(zv, outh.at[pl.ds(c * NPAD + s * SPT, SPT)])

    return k(ei)


def _tc_scales(dego_raw, degi_raw):
    def body(a_ref, b_ref, so_ref, si_ref):
        po = a_ref[pl.ds(0, NPAD)] + a_ref[pl.ds(NPAD, NPAD)]
        pi = b_ref[pl.ds(0, NPAD)] + b_ref[pl.ds(NPAD, NPAD)]
        so = lax.rsqrt(jnp.maximum(po, 1.0))
        si = lax.rsqrt(jnp.maximum(pi, 1.0))
        so_ref[...] = jnp.broadcast_to(so[:, None], (NPAD, D))
        si_ref[...] = jnp.broadcast_to(si[:, None], (NPAD, D))

    return pl.pallas_call(
        body,
        out_shape=(
            jax.ShapeDtypeStruct((NPAD, D), jnp.float32),
            jax.ShapeDtypeStruct((NPAD, D), jnp.float32),
        ),
    )(dego_raw, degi_raw)


def _tc_hscale(h, so2d):
    BR = 1000

    def body(h_ref, s_ref, o_ref):
        o_ref[...] = h_ref[...] * s_ref[...]

    return pl.pallas_call(
        body,
        grid=(N // BR,),
        in_specs=[
            pl.BlockSpec((BR, D), lambda i: (i, 0)),
            pl.BlockSpec((BR, D), lambda i: (i, 0)),
        ],
        out_specs=pl.BlockSpec((BR, D), lambda i: (i, 0)),
        out_shape=jax.ShapeDtypeStruct((N, D), jnp.float32),
    )(h, so2d)


def _tc_matmul(x, W):
    BR = 1000

    def body(x_ref, w_ref, h_ref):
        h_ref[...] = jnp.dot(
            x_ref[...], w_ref[...], preferred_element_type=jnp.float32
        )

    return pl.pallas_call(
        body,
        grid=(N // BR,),
        in_specs=[
            pl.BlockSpec((BR, D), lambda i: (i, 0)),
            pl.BlockSpec((D, D), lambda i: (0, 0)),
        ],
        out_specs=pl.BlockSpec((BR, D), lambda i: (i, 0)),
        out_shape=jax.ShapeDtypeStruct((N, D), jnp.float32),
    )(x, W)


def _sc_gather_scatter(h, ei, w):
    @functools.partial(
        pl.kernel,
        mesh=_mesh(),
        out_type=jax.ShapeDtypeStruct((NC, NPAD, D), jnp.float32),
        scratch_types=[
            pltpu.VMEM_SHARED((NPAD, D), jnp.float32),
            pltpu.VMEM((3 * SUB, D), jnp.float32),
            pltpu.VMEM((4 * SUB,), jnp.int32),
            pltpu.VMEM((4 * SUB,), jnp.int32),
            pltpu.VMEM((4 * SUB,), jnp.float32),
            pltpu.VMEM((SUB, D), jnp.float32),
            pltpu.SemaphoreType.DMA,
            pltpu.SemaphoreType.DMA,
            pltpu.SemaphoreType.DMA,
        ],
    )
    def k(h_h, ei_h, w_h, out_h,
          acc_sh, rows_v, sq, dq, wq, zv, ld_sem, g_sem, sc_sem):
        c = lax.axis_index("c")
        s = lax.axis_index("s")
        wid = s * NC + c
        ebase = wid * EPW

        def fillz(i, carry):
            for cc in range(8):
                zv[i, pl.ds(cc * 16, 16)] = jnp.zeros((16,), jnp.float32)
            return carry

        lax.fori_loop(0, SUB, fillz, 0)

        def fillq(i, carry):
            sq[pl.ds(i * 16, 16)] = jnp.zeros((16,), jnp.int32)
            dq[pl.ds(i * 16, 16)] = jnp.zeros((16,), jnp.int32)
            return carry

        lax.fori_loop(0, (4 * SUB) // 16, fillq, 0)

        zs = [
            pltpu.async_copy(zv, acc_sh.at[pl.ds(s * SPT + j * SUB, SUB)],
                             ld_sem)
            for j in range(SPT // SUB)
        ]
        for z in zs:
            z.wait()
        plsc.subcore_barrier()

        def q4(t):
            return lax.rem(t, 4) * SUB

        def r3(t):
            return lax.rem(t, 3) * SUB

        def lbase(t):
            return ebase + jnp.minimum(t, RPW - 1) * SUB

        def issue_loads(t):
            pltpu.async_copy(ei_h.at[pl.ds(lbase(t), SUB)],
                             sq.at[pl.ds(q4(t), SUB)], ld_sem)
            pltpu.async_copy(ei_h.at[pl.ds(E + lbase(t), SUB)],
                             dq.at[pl.ds(q4(t), SUB)], ld_sem)
            pltpu.async_copy(w_h.at[pl.ds(lbase(t), SUB)],
                             wq.at[pl.ds(q4(t), SUB)], ld_sem)

        def wait_loads(t):
            pltpu.make_async_copy(ei_h.at[pl.ds(lbase(t), SUB)],
                                  sq.at[pl.ds(q4(t), SUB)], ld_sem).wait()
            pltpu.make_async_copy(ei_h.at[pl.ds(E + lbase(t), SUB)],
                                  dq.at[pl.ds(q4(t), SUB)], ld_sem).wait()
            pltpu.make_async_copy(w_h.at[pl.ds(lbase(t), SUB)],
                                  wq.at[pl.ds(q4(t), SUB)], ld_sem).wait()

        def issue_gather(t):
            pltpu.async_copy(h_h.at[sq.at[pl.ds(q4(t), SUB)]],
                             rows_v.at[pl.ds(r3(t), SUB)], g_sem)

        def wait_gather(t):
            pltpu.make_async_copy(h_h.at[sq.at[pl.ds(q4(t), SUB)]],
                                  rows_v.at[pl.ds(r3(t), SUB)], g_sem).wait()

        def wait_scatter(t):
            pltpu.make_async_copy(
                rows_v.at[pl.ds(r3(t), SUB)],
                acc_sh.at[dq.at[pl.ds(q4(t), SUB)]], sc_sem).wait()

        for slot in (2, 3):
            pltpu.async_copy(zv, acc_sh.at[dq.at[pl.ds(slot * SUB, SUB)]],
                             sc_sem, add=True)
        issue_loads(0)
        issue_loads(1)
        wait_loads(0)
        issue_gather(0)

        dnums = lax.GatherDimensionNumbers(
            offset_dims=(), collapsed_slice_dims=(0,), start_index_map=(0,)
        )

        def body(bi, carry):
            pltpu.make_async_copy(
                rows_v.at[pl.ds(r3(bi + 1), SUB)],
                acc_sh.at[dq.at[pl.ds(q4(bi + 2), SUB)]],
                sc_sem).wait()
            wait_loads(bi + 1)
            issue_gather(bi + 1)
            issue_loads(bi + 2)
            wait_gather(bi)
            rb = r3(bi)
            qb = q4(bi)

            def mul(g, carry2):
                wchunk = wq[pl.ds(qb + g * 16, 16)]
                for lane in range(16):
                    wv = lax.gather(
                        wchunk,
                        jnp.full((16, 1), lane, jnp.int32),
                        dnums,
                        (1,),
                        mode=lax.GatherScatterMode.PROMISE_IN_BOUNDS,
                    )
                    jj = rb + g * 16 + lane
                    for cc in range(8):
                        sl = pl.ds(cc * 16, 16)
                        rows_v[jj, sl] = rows_v[jj, sl] * wv
                return carry2

            lax.fori_loop(0, SUB // 16, mul, 0)

            pltpu.async_copy(rows_v.at[pl.ds(rb, SUB)],
                             acc_sh.at[dq.at[pl.ds(qb, SUB)]],
                             sc_sem, add=True)
            return carry

        lax.fori_loop(0, RPW, body, 0)

        for t in (RPW - 2, RPW - 1):
            wait_scatter(t)
        wait_gather(RPW)
        wait_loads(RPW + 1)
        plsc.subcore_barrier()
        for j in range(SPT // 160):
            pltpu.sync_copy(
                acc_sh.at[pl.ds(s * SPT + j * 160, 160)],
                rows_v.at[pl.ds(0, 160)],
            )
            pltpu.sync_copy(
                rows_v.at[pl.ds(0, 160)],
                out_h.at[c, pl.ds(s * SPT + j * 160, 160)],
            )

    return k(h, ei, w)


def _tc_combine(p, si2d, b2):
    BR = 1000

    def body(p_ref, s_ref, b_ref, o_ref):
        o_ref[...] = (p_ref[0] + p_ref[1]) * s_ref[...] + b_ref[...]

    return pl.pallas_call(
        body,
        grid=(N // BR,),
        in_specs=[
            pl.BlockSpec((2, BR, D), lambda i: (0, i, 0)),
            pl.BlockSpec((BR, D), lambda i: (i, 0)),
            pl.BlockSpec((1, D), lambda i: (0, 0)),
        ],
        out_specs=pl.BlockSpec((BR, D), lambda i: (i, 0)),
        out_shape=jax.ShapeDtypeStruct((N, D), jnp.float32),
    )(p, si2d, b2)


def kernel(x, edge_index, edge_weight, W, b):
    ei = edge_index.astype(jnp.int32).reshape(2 * E)
    w = edge_weight.astype(jnp.float32)

    h0 = _tc_matmul(x, W)
    dego_raw, degi_raw = _sc_degrees(ei)
    so2d, si2d = _tc_scales(dego_raw, degi_raw)
    h = _tc_hscale(h0, so2d)
    partials = _sc_gather_scatter(h, ei, w)
    out = _tc_combine(partials, si2d, b.reshape(1, D))
    return out
